# baseline (speedup 1.0000x reference)
import jax
import jax.numpy as jnp
from jax.experimental import pallas as pl

_NUM_CENTROIDS = 1024
_TILE = 512
_CHUNK = 128


def _vq_tile(x_ref, cb_ref, q_ref, loss_ref, idx_ref):
    x = x_ref[...]
    nchunks = _NUM_CENTROIDS // _CHUNK
    best_d = jnp.full((_TILE,), jnp.inf, dtype=jnp.float32)
    best_i = jnp.zeros((_TILE,), dtype=jnp.int32)
    for c in range(nchunks):
        cbc = cb_ref[pl.ds(c * _CHUNK, _CHUNK), :]
        csq = jnp.sum(cbc * cbc, axis=1)
        d = csq[None, :] - 2.0 * jax.lax.dot_general(
            x, cbc, (((1,), (1,)), ((), ())),
            preferred_element_type=jnp.float32)
        d_loc = jnp.min(d, axis=1)
        i_loc = jnp.argmin(d, axis=1).astype(jnp.int32) + c * _CHUNK
        upd = d_loc < best_d
        best_i = jnp.where(upd, i_loc, best_i)
        best_d = jnp.where(upd, d_loc, best_d)
    q = jnp.zeros(x.shape, dtype=jnp.float32)
    for c in range(nchunks):
        cbc = cb_ref[pl.ds(c * _CHUNK, _CHUNK), :]
        onehot = (jax.lax.broadcasted_iota(jnp.int32, (_TILE, _CHUNK), 1)
                  + c * _CHUNK == best_i[:, None]).astype(jnp.float32)
        q = q + jax.lax.dot_general(
            onehot, cbc, (((1,), (0,)), ((), ())),
            preferred_element_type=jnp.float32)
    loss_ref[...] = 1.25 * jnp.square(q - x)
    q_ref[...] = x + (q - x)
    idx_ref[...] = best_i


def kernel(inputs, codebook, cluster_counts, train):
    b, t, d = inputs.shape
    flat = inputs.reshape(-1, d)
    n = flat.shape[0]
    grid = n // _TILE
    q, loss, idx = pl.pallas_call(
        _vq_tile,
        grid=(grid,),
        in_specs=[
            pl.BlockSpec((_TILE, d), lambda i: (i, 0)),
            pl.BlockSpec((_NUM_CENTROIDS, d), lambda i: (0, 0)),
        ],
        out_specs=[
            pl.BlockSpec((_TILE, d), lambda i: (i, 0)),
            pl.BlockSpec((_TILE, d), lambda i: (i, 0)),
            pl.BlockSpec((_TILE,), lambda i: (i,)),
        ],
        out_shape=[
            jax.ShapeDtypeStruct((n, d), jnp.float32),
            jax.ShapeDtypeStruct((n, d), jnp.float32),
            jax.ShapeDtypeStruct((n,), jnp.int32),
        ],
    )(flat, codebook)
    quantized = q.reshape(inputs.shape)
    qloss = loss.reshape(inputs.shape)
    nn_idx = idx.reshape(1, b, t)
    codebook_values = jax.lax.stop_gradient(codebook)[None]
    return (quantized, qloss, nn_idx, codebook_values, cluster_counts)


# speedup vs baseline: 26.7979x; 26.7979x over previous
import functools

import jax
import jax.numpy as jnp
from jax import lax
from jax.experimental import pallas as pl
from jax.experimental.pallas import tpu as pltpu
from jax.experimental.pallas import tpu_sc as plsc

_C = 1024
_D = 64
_TILE_A = 128
_CHUNK = 256
_NCHUNK = _C // _CHUNK
_TILE_C = 1024


def _dist_kernel(x_ref, cb_ref, idx_ref, csq_ref):
    @pl.when(pl.program_id(0) == 0)
    def _():
        for c in range(_NCHUNK):
            cbc = cb_ref[pl.ds(c * _CHUNK, _CHUNK), :]
            csq_ref[:, pl.ds(c * _CHUNK, _CHUNK)] = lax.dot_general(
                jnp.ones((1, _D), jnp.float32), cbc * cbc,
                (((1,), (1,)), ((), ())), preferred_element_type=jnp.float32)

    x = x_ref[...]
    best = None
    chunk_of = None
    for c in range(_NCHUNK):
        cbc = cb_ref[pl.ds(c * _CHUNK, _CHUNK), :]
        m = lax.dot_general(x, cbc, (((1,), (1,)), ((), ())),
                            preferred_element_type=jnp.float32)
        d = csq_ref[:, pl.ds(c * _CHUNK, _CHUNK)] - 2.0 * m
        if best is None:
            best = d
            chunk_of = jnp.zeros(d.shape, jnp.int32)
        else:
            upd = d < best
            best = jnp.where(upd, d, best)
            chunk_of = jnp.where(upd, c, chunk_of)
    lane = lax.broadcasted_iota(jnp.int32, (_TILE_A, _CHUNK), 1)
    mmin = jnp.min(best, axis=1, keepdims=True)
    cand = jnp.where(best == mmin, chunk_of * _CHUNK + lane, jnp.int32(2**30))
    idx_ref[0, 0, :] = jnp.min(cand, axis=1)


def _loss_kernel(x_ref, q_ref, qst_ref, loss_ref):
    x = x_ref[...]
    q = q_ref[:, : x.shape[1]]
    dlt = q - x
    qst_ref[...] = x + dlt
    loss_ref[...] = 1.25 * jnp.square(dlt)


_DPAD = 128


def _make_sc_gather(n_tokens):
    info = plsc.get_sparse_core_info()
    nc, ns = info.num_cores, info.num_subcores
    nw = nc * ns
    b_per_w = n_tokens // nw
    mesh = plsc.VectorSubcoreMesh(core_axis_name="c", subcore_axis_name="s")

    @functools.partial(
        pl.kernel, mesh=mesh,
        out_type=jax.ShapeDtypeStruct((n_tokens, _DPAD), jnp.float32),
        scratch_types=[
            pltpu.VMEM((b_per_w,), jnp.int32),
            pltpu.VMEM((b_per_w, _DPAD), jnp.float32),
            pltpu.SemaphoreType.DMA,
        ],
    )
    def gather(table_hbm, idx_hbm, out_hbm, idx_v, rows_v, sem):
        wid = lax.axis_index("s") * nc + lax.axis_index("c")
        base = wid * b_per_w
        pltpu.sync_copy(idx_hbm.at[pl.ds(base, b_per_w)], idx_v)
        pltpu.async_copy(table_hbm.at[idx_v], rows_v, sem).wait()
        pltpu.sync_copy(rows_v, out_hbm.at[pl.ds(base, b_per_w)])

    return gather


def kernel(inputs, codebook, cluster_counts, train):
    b, t, d = inputs.shape
    flat = inputs.reshape(-1, d)
    n = flat.shape[0]

    idx3 = pl.pallas_call(
        _dist_kernel,
        grid=(n // _TILE_A,),
        in_specs=[
            pl.BlockSpec((_TILE_A, d), lambda i: (i, 0)),
            pl.BlockSpec((_C, d), lambda i: (0, 0)),
        ],
        out_specs=pl.BlockSpec((1, 1, _TILE_A), lambda i: (i, 0, 0)),
        out_shape=jax.ShapeDtypeStruct((n // _TILE_A, 1, _TILE_A), jnp.int32),
        scratch_shapes=[pltpu.VMEM((1, _C), jnp.float32)],
    )(flat, codebook)
    idx = idx3.reshape(n)

    cb_pad = jnp.concatenate(
        [codebook, jnp.zeros((_C, _DPAD - d), jnp.float32)], axis=1)
    q = _make_sc_gather(n)(cb_pad, idx)

    qst, loss = pl.pallas_call(
        _loss_kernel,
        grid=(n // _TILE_C,),
        in_specs=[
            pl.BlockSpec((_TILE_C, d), lambda i: (i, 0)),
            pl.BlockSpec((_TILE_C, _DPAD), lambda i: (i, 0)),
        ],
        out_specs=[
            pl.BlockSpec((_TILE_C, d), lambda i: (i, 0)),
            pl.BlockSpec((_TILE_C, d), lambda i: (i, 0)),
        ],
        out_shape=[
            jax.ShapeDtypeStruct((n, d), jnp.float32),
            jax.ShapeDtypeStruct((n, d), jnp.float32),
        ],
    )(flat, q)

    quantized = qst.reshape(inputs.shape)
    qloss = loss.reshape(inputs.shape)
    nn_idx = idx.reshape(1, b, t)
    codebook_values = jax.lax.stop_gradient(codebook)[None]
    return (quantized, qloss, nn_idx, codebook_values, cluster_counts)


# speedup vs baseline: 34.5192x; 1.2881x over previous
import functools

import jax
import jax.numpy as jnp
from jax import lax
from jax.experimental import pallas as pl
from jax.experimental.pallas import tpu as pltpu
from jax.experimental.pallas import tpu_sc as plsc

_C = 1024
_D = 64
_TILE_A = 256
_CHUNK = 128
_NCHUNK = _C // _CHUNK
_TILE_C = 1024


def _dist_kernel(x_ref, cb_ref, idx_ref, csq_ref):
    @pl.when(pl.program_id(0) == 0)
    def _():
        cb = cb_ref[...]
        csq_ref[...] = jnp.sum(cb * cb, axis=1, keepdims=True)

    x = x_ref[...]
    best_d = None
    best_i = None
    for c in range(_NCHUNK):
        cbc = cb_ref[pl.ds(c * _CHUNK, _CHUNK), :]
        m = lax.dot_general(cbc, x, (((1,), (1,)), ((), ())),
                            preferred_element_type=jnp.float32)
        d = csq_ref[pl.ds(c * _CHUNK, _CHUNK), :] - 2.0 * m
        dmin = jnp.min(d, axis=0, keepdims=True)
        row = lax.broadcasted_iota(jnp.int32, d.shape, 0) + c * _CHUNK
        imin = jnp.min(jnp.where(d == dmin, row, jnp.int32(2**30)),
                       axis=0, keepdims=True)
        if best_d is None:
            best_d, best_i = dmin, imin
        else:
            upd = dmin < best_d
            best_i = jnp.where(upd, imin, best_i)
            best_d = jnp.where(upd, dmin, best_d)
    idx_ref[...] = best_i.reshape(1, 1, _TILE_A)


def _loss_kernel(x_ref, q_ref, qst_ref, loss_ref):
    x = x_ref[...]
    q = q_ref[:, : x.shape[1]]
    dlt = q - x
    qst_ref[...] = x + dlt
    loss_ref[...] = 1.25 * jnp.square(dlt)


_DPAD = 128


def _make_sc_gather(n_tokens):
    info = plsc.get_sparse_core_info()
    nc, ns = info.num_cores, info.num_subcores
    nw = nc * ns
    b_per_w = n_tokens // nw
    mesh = plsc.VectorSubcoreMesh(core_axis_name="c", subcore_axis_name="s")

    @functools.partial(
        pl.kernel, mesh=mesh,
        out_type=jax.ShapeDtypeStruct((n_tokens, _DPAD), jnp.float32),
        scratch_types=[
            pltpu.VMEM((b_per_w,), jnp.int32),
            pltpu.VMEM((b_per_w, _DPAD), jnp.float32),
            pltpu.SemaphoreType.DMA,
        ],
    )
    def gather(table_hbm, idx_hbm, out_hbm, idx_v, rows_v, sem):
        wid = lax.axis_index("s") * nc + lax.axis_index("c")
        base = wid * b_per_w
        pltpu.sync_copy(idx_hbm.at[pl.ds(base, b_per_w)], idx_v)
        pltpu.async_copy(table_hbm.at[idx_v], rows_v, sem).wait()
        pltpu.sync_copy(rows_v, out_hbm.at[pl.ds(base, b_per_w)])

    return gather


def kernel(inputs, codebook, cluster_counts, train):
    b, t, d = inputs.shape
    flat = inputs.reshape(-1, d)
    n = flat.shape[0]

    idx3 = pl.pallas_call(
        _dist_kernel,
        grid=(n // _TILE_A,),
        in_specs=[
            pl.BlockSpec((_TILE_A, d), lambda i: (i, 0)),
            pl.BlockSpec((_C, d), lambda i: (0, 0)),
        ],
        out_specs=pl.BlockSpec((1, 1, _TILE_A), lambda i: (i, 0, 0)),
        out_shape=jax.ShapeDtypeStruct((n // _TILE_A, 1, _TILE_A), jnp.int32),
        scratch_shapes=[pltpu.VMEM((_C, 1), jnp.float32)],
    )(flat, codebook)
    idx = idx3.reshape(n)

    cb_pad = jnp.concatenate(
        [codebook, jnp.zeros((_C, _DPAD - d), jnp.float32)], axis=1)
    q = _make_sc_gather(n)(cb_pad, idx)

    qst, loss = pl.pallas_call(
        _loss_kernel,
        grid=(n // _TILE_C,),
        in_specs=[
            pl.BlockSpec((_TILE_C, d), lambda i: (i, 0)),
            pl.BlockSpec((_TILE_C, _DPAD), lambda i: (i, 0)),
        ],
        out_specs=[
            pl.BlockSpec((_TILE_C, d), lambda i: (i, 0)),
            pl.BlockSpec((_TILE_C, d), lambda i: (i, 0)),
        ],
        out_shape=[
            jax.ShapeDtypeStruct((n, d), jnp.float32),
            jax.ShapeDtypeStruct((n, d), jnp.float32),
        ],
    )(flat, q)

    quantized = qst.reshape(inputs.shape)
    qloss = loss.reshape(inputs.shape)
    nn_idx = idx.reshape(1, b, t)
    codebook_values = jax.lax.stop_gradient(codebook)[None]
    return (quantized, qloss, nn_idx, codebook_values, cluster_counts)
